# baseline (speedup 1.0000x reference)
import jax
import jax.numpy as jnp
import numpy as np
from jax import lax
from jax.experimental import pallas as pl
from jax.experimental.pallas import tpu as pltpu
from jax.experimental.pallas import tpu_sc as plsc

_NUM_CLASSES = 1000
_SMOOTHING = 0.1
_BATCH = 16384

_NUM_WORKERS = 32
_ROWS_PER_WORKER = _BATCH // _NUM_WORKERS
_CHUNK = 128
_NCHUNKS = _ROWS_PER_WORKER // _CHUNK
_LANES = 16
_CHUNK_WORDS = _CHUNK * _NUM_CLASSES
_IDX_ROW = 128
_NSCAT = _ROWS_PER_WORKER // _IDX_ROW

_BASE = float(np.float32(_SMOOTHING / _NUM_CLASSES))
_PEAK = float(np.float32(np.float32(_BASE) + np.float32(1.0 - _SMOOTHING)))


def _sc_body(target_hbm, out_hbm, tgt_v, cbuf, peak_v, idx_v, *sems):
    fill_sems = sems[:_NCHUNKS]
    scat_sems = sems[_NCHUNKS:]
    wid = lax.axis_index("s") * 2 + lax.axis_index("c")
    row0 = wid * _ROWS_PER_WORKER

    base_vec = jnp.full((_LANES,), _BASE, jnp.float32)
    peak_vec = jnp.full((_LANES,), _PEAK, jnp.float32)
    lane_iota = lax.broadcasted_iota(jnp.int32, (_LANES,), 0)

    pltpu.sync_copy(target_hbm.at[pl.ds(row0, _ROWS_PER_WORKER)], tgt_v)

    def fill_grp(j, _):
        for u in range(4):
            cbuf[pl.ds((j * 4 + u) * _LANES, _LANES)] = base_vec
        return 0

    lax.fori_loop(0, _CHUNK_WORDS // (_LANES * 4), fill_grp, 0)

    def fill_peak(j, _):
        peak_v[pl.ds(j * _LANES, _LANES)] = peak_vec
        return 0

    lax.fori_loop(0, _IDX_ROW // _LANES, fill_peak, 0)

    def fill_idx(g, _):
        t = tgt_v[pl.ds(g * _LANES, _LANES)]
        idx = (lane_iota + (row0 + g * _LANES)) * _NUM_CLASSES + t
        idx_v[g // (_IDX_ROW // _LANES), pl.ds((g % (_IDX_ROW // _LANES)) * _LANES, _LANES)] = idx
        return 0

    for g in range(_ROWS_PER_WORKER // _LANES):
        fill_idx(g, 0)

    fills = []
    for c in range(_NCHUNKS):
        fills.append(pltpu.async_copy(
            cbuf,
            out_hbm.at[pl.ds((row0 + c * _CHUNK) * _NUM_CLASSES, _CHUNK_WORDS)],
            fill_sems[c]))
    for f in fills:
        f.wait()

    scats = []
    for s in range(_NSCAT):
        scats.append(pltpu.async_copy(
            peak_v, out_hbm.at[idx_v.at[s]], scat_sems[s]))
    for s in scats:
        s.wait()


@jax.jit
def _sc_call(target):
    mesh = plsc.VectorSubcoreMesh(core_axis_name="c", subcore_axis_name="s")
    flat = pl.kernel(
        _sc_body,
        mesh=mesh,
        compiler_params=pltpu.CompilerParams(needs_layout_passes=False),
        out_type=jax.ShapeDtypeStruct((_BATCH * _NUM_CLASSES,), jnp.float32),
        scratch_types=[
            pltpu.VMEM((_ROWS_PER_WORKER,), jnp.int32),
            pltpu.VMEM((_CHUNK_WORDS,), jnp.float32),
            pltpu.VMEM((_IDX_ROW,), jnp.float32),
            pltpu.VMEM((_NSCAT, _IDX_ROW), jnp.int32),
        ] + [pltpu.SemaphoreType.DMA] * (_NCHUNKS + _NSCAT),
    )(target)
    return flat.reshape(_BATCH, _NUM_CLASSES)


def kernel(target, pred):
    del pred
    return _sc_call(target)


# speedup vs baseline: 1.7422x; 1.7422x over previous
import jax
import jax.numpy as jnp
import numpy as np
from jax import lax
from jax.experimental import pallas as pl
from jax.experimental.pallas import tpu as pltpu
from jax.experimental.pallas import tpu_sc as plsc

_NUM_CLASSES = 1000
_SMOOTHING = 0.1
_BATCH = 16384

_NUM_WORKERS = 32
_ROWS_PER_WORKER = _BATCH // _NUM_WORKERS
_CHUNK = 32
_NBUF = 3
_NCHUNKS = _ROWS_PER_WORKER // _CHUNK
_LANES = 16

_BASE = float(np.float32(_SMOOTHING / _NUM_CLASSES))
_PEAK = float(np.float32(np.float32(_BASE) + np.float32(1.0 - _SMOOTHING)))


def _sc_body(target_hbm, out_hbm, tgt_v, *rest):
    bufs = rest[:_NBUF]
    sems = rest[_NBUF:2 * _NBUF]
    wid = lax.axis_index("s") * 2 + lax.axis_index("c")
    row0 = wid * _ROWS_PER_WORKER

    base_vec = jnp.full((_LANES,), _BASE, jnp.float32)
    peak_vec = jnp.full((_LANES,), _PEAK, jnp.float32)
    lane_iota = lax.broadcasted_iota(jnp.int32, (_LANES,), 0)

    pltpu.sync_copy(target_hbm.at[pl.ds(row0, _ROWS_PER_WORKER)], tgt_v)

    def fill_row(r, _):
        def fill_grp(j, _):
            for b in bufs:
                b[r, pl.ds(j * _LANES, _LANES)] = base_vec
            return 0
        lax.fori_loop(0, _NUM_CLASSES // _LANES, fill_grp, 0)
        for b in bufs:
            b[r, pl.ds(_NUM_CLASSES - _LANES, _LANES)] = base_vec
        return 0

    lax.fori_loop(0, _CHUNK, fill_row, 0)

    copies = [None] * _NBUF
    groups = _CHUNK // _LANES

    for c in range(_NCHUNKS):
        slot = c % _NBUF
        buf = bufs[slot]
        if copies[slot] is not None:
            copies[slot].wait()
            for g in range(groups):
                pt = tgt_v[pl.ds((c - _NBUF) * _CHUNK + g * _LANES, _LANES)]
                plsc.store_scatter(buf, [lane_iota + g * _LANES, pt], base_vec)
        for g in range(groups):
            t = tgt_v[pl.ds(c * _CHUNK + g * _LANES, _LANES)]
            plsc.store_scatter(buf, [lane_iota + g * _LANES, t], peak_vec)
        copies[slot] = pltpu.async_copy(
            buf, out_hbm.at[pl.ds(row0 + c * _CHUNK, _CHUNK)], sems[slot])

    for slot in range(_NBUF):
        copies[slot].wait()


@jax.jit
def _sc_call(target):
    mesh = plsc.VectorSubcoreMesh(core_axis_name="c", subcore_axis_name="s")
    return pl.kernel(
        _sc_body,
        mesh=mesh,
        compiler_params=pltpu.CompilerParams(needs_layout_passes=False),
        out_type=jax.ShapeDtypeStruct((_BATCH, _NUM_CLASSES), jnp.float32),
        scratch_types=[
            pltpu.VMEM((_ROWS_PER_WORKER,), jnp.int32),
        ] + [pltpu.VMEM((_CHUNK, _NUM_CLASSES), jnp.float32)] * _NBUF
          + [pltpu.SemaphoreType.DMA] * _NBUF,
    )(target)


def kernel(target, pred):
    del pred
    return _sc_call(target)


# speedup vs baseline: 3.5471x; 2.0360x over previous
import jax
import jax.numpy as jnp
import numpy as np
from jax import lax
from jax.experimental import pallas as pl
from jax.experimental.pallas import tpu as pltpu
from jax.experimental.pallas import tpu_sc as plsc

_NUM_CLASSES = 1000
_SMOOTHING = 0.1
_BATCH = 16384

_NUM_WORKERS = 32
_ROWS = 32
_COLW = 1024
_NCHUNKS = _BATCH // _COLW
_NBUF = 3
_LANES = 16
_GROUPS = _COLW // _LANES

_BASE = float(np.float32(_SMOOTHING / _NUM_CLASSES))
_PEAK = float(np.float32(np.float32(_BASE) + np.float32(1.0 - _SMOOTHING)))


def _sc_body(target_hbm, out_hbm, tgt_v, *rest):
    bufs = rest[:_NBUF]
    sems = rest[_NBUF:2 * _NBUF]
    wid = lax.axis_index("s") * 2 + lax.axis_index("c")
    r0 = jnp.minimum(wid * _ROWS, _NUM_CLASSES - _ROWS)

    base_vec = jnp.full((_LANES,), _BASE, jnp.float32)
    peak_vec = jnp.full((_LANES,), _PEAK, jnp.float32)
    lane_iota = lax.broadcasted_iota(jnp.int32, (_LANES,), 0)

    pltpu.sync_copy(target_hbm, tgt_v)

    def fill_row(r, _):
        def fill_grp(g, _):
            for b in bufs:
                b[r, pl.ds(g * _LANES, _LANES)] = base_vec
            return 0
        lax.fori_loop(0, _GROUPS, fill_grp, 0)
        return 0

    lax.fori_loop(0, _ROWS, fill_row, 0)

    def scatter_chunk(c, buf, value_vec):
        def grp(g, _):
            t = tgt_v[pl.ds(c * _COLW + g * _LANES, _LANES)]
            rows = t - r0
            mask = (t >= r0) & (t < r0 + _ROWS)
            plsc.store_scatter(buf, [rows, lane_iota + g * _LANES], value_vec,
                               mask=mask)
            return 0
        lax.fori_loop(0, _GROUPS, grp, 0)

    copies = [None] * _NBUF
    for c in range(_NCHUNKS):
        slot = c % _NBUF
        buf = bufs[slot]
        if copies[slot] is not None:
            copies[slot].wait()
            scatter_chunk(c - _NBUF, buf, base_vec)
        scatter_chunk(c, buf, peak_vec)
        copies[slot] = pltpu.async_copy(
            buf, out_hbm.at[pl.ds(r0, _ROWS), pl.ds(c * _COLW, _COLW)],
            sems[slot])

    for slot in range(_NBUF):
        copies[slot].wait()


@jax.jit
def _sc_call(target):
    mesh = plsc.VectorSubcoreMesh(core_axis_name="c", subcore_axis_name="s")
    q_t = pl.kernel(
        _sc_body,
        mesh=mesh,
        compiler_params=pltpu.CompilerParams(needs_layout_passes=False),
        out_type=jax.ShapeDtypeStruct((_NUM_CLASSES, _BATCH), jnp.float32),
        scratch_types=[
            pltpu.VMEM((_BATCH,), jnp.int32),
        ] + [pltpu.VMEM((_ROWS, _COLW), jnp.float32)] * _NBUF
          + [pltpu.SemaphoreType.DMA] * _NBUF,
    )(target)
    return q_t.T


def kernel(target, pred):
    del pred
    return _sc_call(target)
